# baseline (speedup 1.0000x reference)
import functools

import jax
import jax.numpy as jnp
from jax import lax
from jax.experimental import pallas as pl
from jax.experimental.pallas import tpu as pltpu
from jax.experimental.pallas import tpu_sc as plsc

B, S, D = 4, 8192, 1024
NW = 32
S_PER_W = S // NW
R = 8
NCH = S_PER_W // R
VPR = D // 16

_mesh = plsc.VectorSubcoreMesh(core_axis_name="c", subcore_axis_name="s")


@functools.partial(
    pl.kernel,
    mesh=_mesh,
    out_type=jax.ShapeDtypeStruct((B, S, D), jnp.float32),
    scratch_types=(
        [pltpu.VMEM((R, D), jnp.float32)] * 2
        + [pltpu.VMEM((R, D), jnp.float32)] * 8
        + [pltpu.SemaphoreType.DMA]
        + [pltpu.SemaphoreType.DMA] * 8
        + [pltpu.SemaphoreType.DMA] * 8
    ),
)
def _sc_add(x_hbm, t_hbm, o_hbm, t0, t1,
            q0, q1, q2, q3, q4, q5, q6, q7,
            st,
            sx0, sx1, sx2, sx3, sx4, sx5, sx6, sx7,
            so0, so1, so2, so3, so4, so5, so6, so7):
    wid = lax.axis_index("c") * 16 + lax.axis_index("s")
    s_base = wid * S_PER_W
    qs = (q0, q1, q2, q3, q4, q5, q6, q7)
    sxs = (sx0, sx1, sx2, sx3, sx4, sx5, sx6, sx7)
    sos = (so0, so1, so2, so3, so4, so5, so6, so7)

    def t_sl(ci):
        return t_hbm.at[pl.ds(s_base + ci * R, R)]

    def x_sl(b, ci):
        return x_hbm.at[b, pl.ds(s_base + ci * R, R)]

    def o_sl(b, ci):
        return o_hbm.at[b, pl.ds(s_base + ci * R, R)]

    def so_drain(q):
        pltpu.make_async_copy(qs[q], o_hbm.at[0, pl.ds(0, R)], sos[q]).wait()

    def do_chunk(ci, tcur, tnext, par, first_pred=None, last_pred=None):
        pltpu.make_async_copy(t_sl(ci), tcur, st).wait()
        if last_pred is None:
            pltpu.async_copy(t_sl(ci + 1), tnext, st)
        else:
            def _tn():
                pltpu.async_copy(t_sl(ci + 1), tnext, st)
            pl.when(jnp.logical_not(last_pred))(_tn)
        for b in range(B):
            q = b + par
            qn = (q + 4) % 8
            xb = qs[q]
            pltpu.make_async_copy(x_sl(b, ci), xb, sxs[q]).wait()
            if first_pred is None:
                so_drain(qn)
            else:
                pl.when(jnp.logical_not(first_pred))(
                    functools.partial(so_drain, qn))
            if last_pred is None:
                pltpu.async_copy(x_sl(b, ci + 1), qs[qn], sxs[qn])
            else:
                def _xn():
                    pltpu.async_copy(x_sl(b, ci + 1), qs[qn], sxs[qn])
                pl.when(jnp.logical_not(last_pred))(_xn)

            def row(r, c2):
                @plsc.parallel_loop(0, VPR, unroll=8)
                def _vec(cv):
                    col = cv * 16
                    v = tcur[r, pl.ds(col, 16)]
                    plsc.addupdate(xb.at[r, pl.ds(col, 16)], v)
                return c2

            lax.fori_loop(0, R, row, 0)
            pltpu.async_copy(xb, o_sl(b, ci), sos[q])

    pltpu.async_copy(t_sl(0), t0, st)
    for b in range(B):
        pltpu.async_copy(x_sl(b, 0), qs[b], sxs[b])

    def pair(j, carry):
        ci = 2 * j
        do_chunk(ci, t0, t1, 0, first_pred=j == 0)
        do_chunk(ci + 1, t1, t0, 4, last_pred=j == (NCH // 2 - 1))
        return carry

    lax.fori_loop(0, NCH // 2, pair, 0)

    so_drain(4)
    so_drain(5)
    so_drain(6)
    so_drain(7)


def kernel(x, embedding_table):
    return _sc_add(x, embedding_table)
